# baseline (speedup 1.0000x reference)
import functools

import numpy as np
import jax
import jax.numpy as jnp
from jax.experimental import pallas as pl
from jax.experimental.pallas import tpu as pltpu

_N = 4096
_D = 16
_K = 4
_CUTOFF = 0.5
_NZ_CUTOFF = 1.4

_B = 256
_CJ = 1024
_BG = 1024

_K1 = np.uint32(0x375F238F)
_K2 = np.uint32(0xCDDB151D)
_TINY = np.float32(np.finfo(np.float32).tiny)
_SCALE = np.float32(np.float32(1.0) - _TINY)

_INTERPRET = False


def _anchor_positive_np():
    i = np.arange(_N, dtype=np.int32)
    a_idx = np.repeat(i, _K - 1)
    pat = np.array([[1, 2, 3], [0, 2, 3], [0, 1, 3], [0, 1, 2]], dtype=np.int32)
    p_idx = (4 * (i[:, None] // 4) + pat[i % 4]).reshape(-1)
    return a_idx, p_idx

_A_IDX_NP, _P_IDX_NP = _anchor_positive_np()


def _dist_lw(xb, xT):
    sim = jnp.dot(xb, xT, preferred_element_type=jnp.float32)
    dist = jnp.sqrt(jnp.maximum(2.0 - 2.0 * sim, 0.0))
    dist = jnp.maximum(dist, _CUTOFF)
    lw = (2.0 - float(_D)) * jnp.log(dist) \
        - (float(_D - 3) / 2.0) * jnp.log(1.0 - 0.25 * dist * dist)
    return dist, lw


def _maxlw_kernel(xb_ref, xT_ref, out_ref):
    _, lw = _dist_lw(xb_ref[...], xT_ref[...])
    out_ref[0, 0, :] = jnp.broadcast_to(jnp.max(lw), (128,))


def _rotl(x, r):
    return (x << np.uint32(r)) | (x >> np.uint32(32 - r))


def _threefry_bits(lin):
    ks0, ks1 = _K1, _K2
    ks2 = np.uint32(ks0 ^ ks1 ^ np.uint32(0x1BD11BDA))
    x0 = jnp.full_like(lin, ks0)
    x1 = lin + ks1

    def rnd(x0, x1, r):
        x0 = x0 + x1
        x1 = _rotl(x1, r) ^ x0
        return x0, x1

    for r in (13, 15, 26, 6):
        x0, x1 = rnd(x0, x1, r)
    x0 = x0 + ks1
    x1 = x1 + np.uint32(ks2 + np.uint32(1))
    for r in (17, 29, 16, 24):
        x0, x1 = rnd(x0, x1, r)
    x0 = x0 + ks2
    x1 = x1 + np.uint32(ks0 + np.uint32(2))
    for r in (13, 15, 26, 6):
        x0, x1 = rnd(x0, x1, r)
    x0 = x0 + ks0
    x1 = x1 + np.uint32(ks1 + np.uint32(3))
    for r in (17, 29, 16, 24):
        x0, x1 = rnd(x0, x1, r)
    x0 = x0 + ks1
    x1 = x1 + np.uint32(ks2 + np.uint32(4))
    for r in (13, 15, 26, 6):
        x0, x1 = rnd(x0, x1, r)
    x0 = x0 + ks2
    x1 = x1 + np.uint32(ks0 + np.uint32(5))
    return x0 ^ x1


def _gumbel_from_bits(bits):
    fb = jax.lax.bitcast_convert_type(
        (bits >> np.uint32(9)) | np.uint32(0x3F800000), jnp.float32)
    floats = fb - np.float32(1.0)
    u = jnp.maximum(_TINY, floats * _SCALE + _TINY)
    return -jnp.log(-jnp.log(u))


def _sample_kernel(m_ref, xf_ref, xbT_ref, o0_ref, o1_ref, o2_ref, lg_ref):
    g = pl.program_id(0)
    r0 = g * _B
    sim = jnp.dot(xf_ref[...], xbT_ref[...], preferred_element_type=jnp.float32)
    dist = jnp.sqrt(jnp.maximum(2.0 - 2.0 * sim, 0.0))
    dist = jnp.maximum(dist, _CUTOFF)
    lw = (2.0 - float(_D)) * jnp.log(dist) \
        - (float(_D - 3) / 2.0) * jnp.log(1.0 - 0.25 * dist * dist)
    m = m_ref[0, 0]
    w = jnp.exp(lw - m)

    jj = jax.lax.broadcasted_iota(jnp.int32, (_N, _B), 0)
    ii = r0 + jax.lax.broadcasted_iota(jnp.int32, (_N, _B), 1)
    maskf = jnp.where((ii // _K) == (jj // _K),
                      np.float32(0.0), np.float32(1.0))
    w = w * maskf
    w = w * (dist < _NZ_CUTOFF).astype(jnp.float32)
    w = jnp.where(w != w, np.float32(0.0), w)
    lg_ref[...] = w

    def racc(t, acc):
        for u in range(8):
            acc = acc + lg_ref[pl.ds(t * 64 + u * 8, 8), :]
        return acc

    acc8 = lg_ref[pl.ds(0, 8), :]
    for u in range(1, 8):
        acc8 = acc8 + lg_ref[pl.ds(u * 8, 8), :]
    acc8 = jax.lax.fori_loop(1, _N // 64, racc, acc8)
    v1 = acc8[0:4, :] + acc8[4:8, :]
    v2 = v1[0:2, :] + v1[2:4, :]
    rs = v2[0:1, :] + v2[1:2, :]

    valid = (rs > 0.0) & (jnp.abs(rs) < jnp.inf)
    probs = lg_ref[...] / rs
    probs = jnp.where(valid, probs, np.float32(1.0 / _N))
    probs = jnp.where(probs != probs, np.float32(1.0 / _N), probs)
    lg_ref[...] = jnp.log(jnp.maximum(probs, np.float32(1e-30)))

    iic = (r0 + jax.lax.broadcasted_iota(jnp.int32, (_CJ, _B), 1)).astype(jnp.uint32)
    jjc = jax.lax.broadcasted_iota(jnp.int32, (_CJ, _B), 0).astype(jnp.uint32)
    basec = iic * np.uint32(_N) + jjc
    out_refs = (o0_ref, o1_ref, o2_ref)
    nchunks = _N // _CJ
    for s in range(3):
        def body(c, carry):
            rmax, ridx = carry
            c0 = c * _CJ
            lin = basec + (np.uint32(s * _N * _N) + c0.astype(jnp.uint32))
            gum = _gumbel_from_bits(_threefry_bits(lin))
            vals = gum + lg_ref[pl.ds(c0, _CJ), :]
            cmax = jnp.max(vals, axis=0, keepdims=True)
            jidx = c0 + jax.lax.broadcasted_iota(jnp.int32, (_CJ, _B), 0)
            cand = jnp.where(vals == cmax, jidx, _N)
            cidx = jnp.min(cand, axis=0, keepdims=True)
            upd = cmax > rmax
            return (jnp.where(upd, cmax, rmax), jnp.where(upd, cidx, ridx))

        rmax0 = jnp.full((1, _B), -jnp.inf, jnp.float32)
        ridx0 = jnp.zeros((1, _B), jnp.int32)
        _, ridx = jax.lax.fori_loop(0, nchunks, body, (rmax0, ridx0))
        out_refs[s][...] = ridx


_NWORK = 32
_NTOT = 3 * _N * (_K - 1)
_BPW = _NTOT // _NWORK


def _sc_gather(idx, table):
    from jax.experimental.pallas import tpu_sc as plsc
    mesh = plsc.VectorSubcoreMesh(core_axis_name="c", subcore_axis_name="s")

    @functools.partial(
        pl.kernel, mesh=mesh,
        out_type=jax.ShapeDtypeStruct((_NTOT, _D), jnp.float32),
        scratch_types=[
            pltpu.VMEM((_BPW,), jnp.int32),
            pltpu.VMEM((_BPW, _D), jnp.float32),
            pltpu.SemaphoreType.DMA,
        ],
        compiler_params=pltpu.CompilerParams(use_tc_tiling_on_sc=False),
    )
    def gk(idx_hbm, table_hbm, out_hbm, idx_v, rows_v, sem):
        wid = jax.lax.axis_index("s") * 2 + jax.lax.axis_index("c")
        base = wid * _BPW
        pltpu.sync_copy(idx_hbm.at[pl.ds(base, _BPW)], idx_v)
        pltpu.async_copy(table_hbm.at[idx_v], rows_v, sem).wait()
        pltpu.sync_copy(rows_v, out_hbm.at[pl.ds(base, _BPW)])

    return gk(idx, table)


def kernel(x):
    xT = x.T
    grid = _N // _B

    maxes = pl.pallas_call(
        _maxlw_kernel,
        grid=(grid,),
        in_specs=[
            pl.BlockSpec((_B, _D), lambda g: (g, 0)),
            pl.BlockSpec((_D, _N), lambda g: (0, 0)),
        ],
        out_specs=pl.BlockSpec((1, 1, 128), lambda g: (g, 0, 0)),
        out_shape=jax.ShapeDtypeStruct((grid, 1, 128), jnp.float32),
        interpret=_INTERPRET,
    )(x, xT)
    m = jnp.max(maxes).reshape(1, 1)

    idx_out = jax.ShapeDtypeStruct((1, _N), jnp.int32)
    o0, o1, o2 = pl.pallas_call(
        _sample_kernel,
        grid=(grid,),
        in_specs=[
            pl.BlockSpec(memory_space=pltpu.SMEM),
            pl.BlockSpec((_N, _D), lambda g: (0, 0)),
            pl.BlockSpec((_D, _B), lambda g: (0, g)),
        ],
        out_specs=[pl.BlockSpec((1, _B), lambda g: (0, g))] * 3,
        out_shape=[idx_out] * 3,
        scratch_shapes=[pltpu.VMEM((_N, _B), jnp.float32)],
        interpret=_INTERPRET,
    )(m, x, xT)
    n_idx = jnp.concatenate([o0, o1, o2], axis=0).T.reshape(-1)

    a_idx = jnp.asarray(_A_IDX_NP)
    idx_all = jnp.concatenate([a_idx, jnp.asarray(_P_IDX_NP), n_idx])
    rows = _sc_gather(idx_all, x)

    nt = _N * (_K - 1)
    a_x = rows[:nt]
    p_x = rows[nt:2 * nt]
    n_x = rows[2 * nt:]
    return (a_idx, a_x, p_x, n_x, x)


# speedup vs baseline: 1.0379x; 1.0379x over previous
import functools

import numpy as np
import jax
import jax.numpy as jnp
from jax.experimental import pallas as pl
from jax.experimental.pallas import tpu as pltpu

_N = 4096
_D = 16
_K = 4
_CUTOFF = 0.5
_NZ_CUTOFF = 1.4

_B = 256
_CJ = 512
_BG = 1024

_K1 = np.uint32(0x375F238F)
_K2 = np.uint32(0xCDDB151D)
_TINY = np.float32(np.finfo(np.float32).tiny)
_SCALE = np.float32(np.float32(1.0) - _TINY)

_INTERPRET = False


def _anchor_positive_np():
    i = np.arange(_N, dtype=np.int32)
    a_idx = np.repeat(i, _K - 1)
    pat = np.array([[1, 2, 3], [0, 2, 3], [0, 1, 3], [0, 1, 2]], dtype=np.int32)
    p_idx = (4 * (i[:, None] // 4) + pat[i % 4]).reshape(-1)
    return a_idx, p_idx

_A_IDX_NP, _P_IDX_NP = _anchor_positive_np()


def _dist_lw(xb, xT):
    sim = jnp.dot(xb, xT, preferred_element_type=jnp.float32)
    dist = jnp.sqrt(jnp.maximum(2.0 - 2.0 * sim, 0.0))
    dist = jnp.maximum(dist, _CUTOFF)
    lw = (2.0 - float(_D)) * jnp.log(dist) \
        - (float(_D - 3) / 2.0) * jnp.log(1.0 - 0.25 * dist * dist)
    return dist, lw


def _maxlw_kernel(xb_ref, xT_ref, out_ref):
    _, lw = _dist_lw(xb_ref[...], xT_ref[...])
    out_ref[0, 0, :] = jnp.broadcast_to(jnp.max(lw), (128,))


def _rotl(x, r):
    return (x << np.uint32(r)) | (x >> np.uint32(32 - r))


def _threefry_bits(lin):
    ks0, ks1 = _K1, _K2
    ks2 = np.uint32(ks0 ^ ks1 ^ np.uint32(0x1BD11BDA))
    x0 = jnp.full_like(lin, ks0)
    x1 = lin + ks1

    def rnd(x0, x1, r):
        x0 = x0 + x1
        x1 = _rotl(x1, r) ^ x0
        return x0, x1

    for r in (13, 15, 26, 6):
        x0, x1 = rnd(x0, x1, r)
    x0 = x0 + ks1
    x1 = x1 + np.uint32(ks2 + np.uint32(1))
    for r in (17, 29, 16, 24):
        x0, x1 = rnd(x0, x1, r)
    x0 = x0 + ks2
    x1 = x1 + np.uint32(ks0 + np.uint32(2))
    for r in (13, 15, 26, 6):
        x0, x1 = rnd(x0, x1, r)
    x0 = x0 + ks0
    x1 = x1 + np.uint32(ks1 + np.uint32(3))
    for r in (17, 29, 16, 24):
        x0, x1 = rnd(x0, x1, r)
    x0 = x0 + ks1
    x1 = x1 + np.uint32(ks2 + np.uint32(4))
    for r in (13, 15, 26, 6):
        x0, x1 = rnd(x0, x1, r)
    x0 = x0 + ks2
    x1 = x1 + np.uint32(ks0 + np.uint32(5))
    return x0 ^ x1


def _gumbel_from_bits(bits):
    fb = jax.lax.bitcast_convert_type(
        (bits >> np.uint32(9)) | np.uint32(0x3F800000), jnp.float32)
    floats = fb - np.float32(1.0)
    u = jnp.maximum(_TINY, floats * _SCALE + _TINY)
    return -jnp.log(-jnp.log(u))


def _sample_kernel(m_ref, xf_ref, xbT_ref, o0_ref, o1_ref, o2_ref, lg_ref):
    g = pl.program_id(0)
    r0 = g * _B
    sim = jnp.dot(xf_ref[...], xbT_ref[...], preferred_element_type=jnp.float32)
    dist = jnp.sqrt(jnp.maximum(2.0 - 2.0 * sim, 0.0))
    dist = jnp.maximum(dist, _CUTOFF)
    lw = (2.0 - float(_D)) * jnp.log(dist) \
        - (float(_D - 3) / 2.0) * jnp.log(1.0 - 0.25 * dist * dist)
    m = m_ref[0, 0]
    w = jnp.exp(lw - m)

    jj = jax.lax.broadcasted_iota(jnp.int32, (_N, _B), 0)
    ii = r0 + jax.lax.broadcasted_iota(jnp.int32, (_N, _B), 1)
    maskf = jnp.where((ii // _K) == (jj // _K),
                      np.float32(0.0), np.float32(1.0))
    w = w * maskf
    w = w * (dist < _NZ_CUTOFF).astype(jnp.float32)
    w = jnp.where(w != w, np.float32(0.0), w)
    lg_ref[...] = w

    def racc(t, acc):
        for u in range(8):
            acc = acc + lg_ref[pl.ds(t * 64 + u * 8, 8), :]
        return acc

    acc8 = lg_ref[pl.ds(0, 8), :]
    for u in range(1, 8):
        acc8 = acc8 + lg_ref[pl.ds(u * 8, 8), :]
    acc8 = jax.lax.fori_loop(1, _N // 64, racc, acc8)
    v1 = acc8[0:4, :] + acc8[4:8, :]
    v2 = v1[0:2, :] + v1[2:4, :]
    rs = v2[0:1, :] + v2[1:2, :]

    valid = (rs > 0.0) & (jnp.abs(rs) < jnp.inf)
    probs = lg_ref[...] / rs
    probs = jnp.where(valid, probs, np.float32(1.0 / _N))
    probs = jnp.where(probs != probs, np.float32(1.0 / _N), probs)
    lg_ref[...] = jnp.log(jnp.maximum(probs, np.float32(1e-30)))

    iic = (r0 + jax.lax.broadcasted_iota(jnp.int32, (_CJ, _B), 1)).astype(jnp.uint32)
    jjc = jax.lax.broadcasted_iota(jnp.int32, (_CJ, _B), 0).astype(jnp.uint32)
    basec = iic * np.uint32(_N) + jjc
    out_refs = (o0_ref, o1_ref, o2_ref)
    nchunks = _N // _CJ
    for s in range(3):
        def body(c, carry):
            rmax, ridx = carry
            c0 = c * _CJ
            lin = basec + (np.uint32(s * _N * _N) + c0.astype(jnp.uint32))
            gum = _gumbel_from_bits(_threefry_bits(lin))
            vals = gum + lg_ref[pl.ds(c0, _CJ), :]
            cmax = jnp.max(vals, axis=0, keepdims=True)
            jidx = c0 + jax.lax.broadcasted_iota(jnp.int32, (_CJ, _B), 0)
            cand = jnp.where(vals == cmax, jidx, _N)
            cidx = jnp.min(cand, axis=0, keepdims=True)
            upd = cmax > rmax
            return (jnp.where(upd, cmax, rmax), jnp.where(upd, cidx, ridx))

        rmax0 = jnp.full((1, _B), -jnp.inf, jnp.float32)
        ridx0 = jnp.zeros((1, _B), jnp.int32)
        _, ridx = jax.lax.fori_loop(0, nchunks, body, (rmax0, ridx0))
        out_refs[s][...] = ridx


_NWORK = 32
_NTOT = 3 * _N * (_K - 1)
_BPW = _NTOT // _NWORK


def _sc_gather(idx, table):
    from jax.experimental.pallas import tpu_sc as plsc
    mesh = plsc.VectorSubcoreMesh(core_axis_name="c", subcore_axis_name="s")

    @functools.partial(
        pl.kernel, mesh=mesh,
        out_type=jax.ShapeDtypeStruct((_NTOT, _D), jnp.float32),
        scratch_types=[
            pltpu.VMEM((_BPW,), jnp.int32),
            pltpu.VMEM((_BPW, _D), jnp.float32),
            pltpu.SemaphoreType.DMA,
        ],
        compiler_params=pltpu.CompilerParams(use_tc_tiling_on_sc=False),
    )
    def gk(idx_hbm, table_hbm, out_hbm, idx_v, rows_v, sem):
        wid = jax.lax.axis_index("s") * 2 + jax.lax.axis_index("c")
        base = wid * _BPW
        pltpu.sync_copy(idx_hbm.at[pl.ds(base, _BPW)], idx_v)
        pltpu.async_copy(table_hbm.at[idx_v], rows_v, sem).wait()
        pltpu.sync_copy(rows_v, out_hbm.at[pl.ds(base, _BPW)])

    return gk(idx, table)


def kernel(x):
    xT = x.T
    grid = _N // _B

    maxes = pl.pallas_call(
        _maxlw_kernel,
        grid=(grid,),
        in_specs=[
            pl.BlockSpec((_B, _D), lambda g: (g, 0)),
            pl.BlockSpec((_D, _N), lambda g: (0, 0)),
        ],
        out_specs=pl.BlockSpec((1, 1, 128), lambda g: (g, 0, 0)),
        out_shape=jax.ShapeDtypeStruct((grid, 1, 128), jnp.float32),
        interpret=_INTERPRET,
    )(x, xT)
    m = jnp.max(maxes).reshape(1, 1)

    idx_out = jax.ShapeDtypeStruct((1, _N), jnp.int32)
    o0, o1, o2 = pl.pallas_call(
        _sample_kernel,
        grid=(grid,),
        in_specs=[
            pl.BlockSpec(memory_space=pltpu.SMEM),
            pl.BlockSpec((_N, _D), lambda g: (0, 0)),
            pl.BlockSpec((_D, _B), lambda g: (0, g)),
        ],
        out_specs=[pl.BlockSpec((1, _B), lambda g: (0, g))] * 3,
        out_shape=[idx_out] * 3,
        scratch_shapes=[pltpu.VMEM((_N, _B), jnp.float32)],
        interpret=_INTERPRET,
    )(m, x, xT)
    n_idx = jnp.concatenate([o0, o1, o2], axis=0).T.reshape(-1)

    a_idx = jnp.asarray(_A_IDX_NP)
    idx_all = jnp.concatenate([a_idx, jnp.asarray(_P_IDX_NP), n_idx])
    rows = _sc_gather(idx_all, x)

    nt = _N * (_K - 1)
    a_x = rows[:nt]
    p_x = rows[nt:2 * nt]
    n_x = rows[2 * nt:]
    return (a_idx, a_x, p_x, n_x, x)


# speedup vs baseline: 1.0492x; 1.0109x over previous
import functools

import numpy as np
import jax
import jax.numpy as jnp
from jax.experimental import pallas as pl
from jax.experimental.pallas import tpu as pltpu

_N = 4096
_D = 16
_K = 4
_CUTOFF = 0.5
_NZ_CUTOFF = 1.4

_B = 256
_CJ = 512
_BG = 1024

_K1 = np.uint32(0x375F238F)
_K2 = np.uint32(0xCDDB151D)
_TINY = np.float32(np.finfo(np.float32).tiny)
_SCALE = np.float32(np.float32(1.0) - _TINY)

_INTERPRET = False


def _anchor_positive_np():
    i = np.arange(_N, dtype=np.int32)
    a_idx = np.repeat(i, _K - 1)
    pat = np.array([[1, 2, 3], [0, 2, 3], [0, 1, 3], [0, 1, 2]], dtype=np.int32)
    p_idx = (4 * (i[:, None] // 4) + pat[i % 4]).reshape(-1)
    return a_idx, p_idx

_A_IDX_NP, _P_IDX_NP = _anchor_positive_np()


def _dist_lw(xb, xT):
    sim = jnp.dot(xb, xT, preferred_element_type=jnp.float32)
    dist = jnp.sqrt(jnp.maximum(2.0 - 2.0 * sim, 0.0))
    dist = jnp.maximum(dist, _CUTOFF)
    lw = (2.0 - float(_D)) * jnp.log(dist) \
        - (float(_D - 3) / 2.0) * jnp.log(1.0 - 0.25 * dist * dist)
    return dist, lw


def _maxlw_kernel(xb_ref, xT_ref, out_ref):
    _, lw = _dist_lw(xb_ref[...], xT_ref[...])
    out_ref[0, 0, :] = jnp.broadcast_to(jnp.max(lw), (128,))


def _rotl(x, r):
    return (x << np.uint32(r)) | (x >> np.uint32(32 - r))


def _threefry_bits(lin):
    ks0, ks1 = _K1, _K2
    ks2 = np.uint32(ks0 ^ ks1 ^ np.uint32(0x1BD11BDA))
    x0 = jnp.full_like(lin, ks0)
    x1 = lin + ks1

    def rnd(x0, x1, r):
        x0 = x0 + x1
        x1 = _rotl(x1, r) ^ x0
        return x0, x1

    for r in (13, 15, 26, 6):
        x0, x1 = rnd(x0, x1, r)
    x0 = x0 + ks1
    x1 = x1 + np.uint32(ks2 + np.uint32(1))
    for r in (17, 29, 16, 24):
        x0, x1 = rnd(x0, x1, r)
    x0 = x0 + ks2
    x1 = x1 + np.uint32(ks0 + np.uint32(2))
    for r in (13, 15, 26, 6):
        x0, x1 = rnd(x0, x1, r)
    x0 = x0 + ks0
    x1 = x1 + np.uint32(ks1 + np.uint32(3))
    for r in (17, 29, 16, 24):
        x0, x1 = rnd(x0, x1, r)
    x0 = x0 + ks1
    x1 = x1 + np.uint32(ks2 + np.uint32(4))
    for r in (13, 15, 26, 6):
        x0, x1 = rnd(x0, x1, r)
    x0 = x0 + ks2
    x1 = x1 + np.uint32(ks0 + np.uint32(5))
    return x0 ^ x1


def _gumbel_from_bits(bits):
    fb = jax.lax.bitcast_convert_type(
        (bits >> np.uint32(9)) | np.uint32(0x3F800000), jnp.float32)
    floats = fb - np.float32(1.0)
    u = floats + _TINY
    return -jnp.log(-jnp.log(u))


def _sample_kernel(m_ref, xf_ref, xbT_ref, o0_ref, o1_ref, o2_ref, lg_ref):
    g = pl.program_id(0)
    r0 = g * _B
    sim = jnp.dot(xf_ref[...], xbT_ref[...], preferred_element_type=jnp.float32)
    dist = jnp.sqrt(jnp.maximum(2.0 - 2.0 * sim, 0.0))
    dist = jnp.maximum(dist, _CUTOFF)
    lw = (2.0 - float(_D)) * jnp.log(dist) \
        - (float(_D - 3) / 2.0) * jnp.log(1.0 - 0.25 * dist * dist)
    m = m_ref[0, 0]
    w = jnp.exp(lw - m)

    jj = jax.lax.broadcasted_iota(jnp.int32, (_N, _B), 0)
    ii = r0 + jax.lax.broadcasted_iota(jnp.int32, (_N, _B), 1)
    maskf = jnp.where((ii // _K) == (jj // _K),
                      np.float32(0.0), np.float32(1.0))
    w = w * maskf
    w = w * (dist < _NZ_CUTOFF).astype(jnp.float32)
    w = jnp.where(w != w, np.float32(0.0), w)
    lg_ref[...] = w

    def racc(t, acc):
        for u in range(8):
            acc = acc + lg_ref[pl.ds(t * 64 + u * 8, 8), :]
        return acc

    acc8 = lg_ref[pl.ds(0, 8), :]
    for u in range(1, 8):
        acc8 = acc8 + lg_ref[pl.ds(u * 8, 8), :]
    acc8 = jax.lax.fori_loop(1, _N // 64, racc, acc8)
    v1 = acc8[0:4, :] + acc8[4:8, :]
    v2 = v1[0:2, :] + v1[2:4, :]
    rs = v2[0:1, :] + v2[1:2, :]

    valid = (rs > 0.0) & (jnp.abs(rs) < jnp.inf)
    probs = lg_ref[...] / rs
    probs = jnp.where(valid, probs, np.float32(1.0 / _N))
    probs = jnp.where(probs != probs, np.float32(1.0 / _N), probs)
    lg_ref[...] = jnp.log(jnp.maximum(probs, np.float32(1e-30)))

    iic = (r0 + jax.lax.broadcasted_iota(jnp.int32, (_CJ, _B), 1)).astype(jnp.uint32)
    jjc = jax.lax.broadcasted_iota(jnp.int32, (_CJ, _B), 0).astype(jnp.uint32)
    basec = iic * np.uint32(_N) + jjc
    out_refs = (o0_ref, o1_ref, o2_ref)
    nchunks = _N // _CJ
    for s in range(3):
        def body(c, carry):
            rmax, ridx = carry
            c0 = c * _CJ
            lin = basec + (np.uint32(s * _N * _N) + c0.astype(jnp.uint32))
            gum = _gumbel_from_bits(_threefry_bits(lin))
            vals = gum + lg_ref[pl.ds(c0, _CJ), :]
            cmax = jnp.max(vals, axis=0, keepdims=True)
            jidx = c0 + jax.lax.broadcasted_iota(jnp.int32, (_CJ, _B), 0)
            cand = jnp.where(vals == cmax, jidx, _N)
            cidx = jnp.min(cand, axis=0, keepdims=True)
            upd = cmax > rmax
            return (jnp.where(upd, cmax, rmax), jnp.where(upd, cidx, ridx))

        rmax0 = jnp.full((1, _B), -jnp.inf, jnp.float32)
        ridx0 = jnp.zeros((1, _B), jnp.int32)
        _, ridx = jax.lax.fori_loop(0, nchunks, body, (rmax0, ridx0))
        out_refs[s][...] = ridx


_NWORK = 32


def _sc_gather(idx, table, ntot):
    from jax.experimental.pallas import tpu_sc as plsc
    mesh = plsc.VectorSubcoreMesh(core_axis_name="c", subcore_axis_name="s")
    bpw = ntot // _NWORK

    @functools.partial(
        pl.kernel, mesh=mesh,
        out_type=jax.ShapeDtypeStruct((ntot, _D), jnp.float32),
        scratch_types=[
            pltpu.VMEM((bpw,), jnp.int32),
            pltpu.VMEM((bpw, _D), jnp.float32),
            pltpu.SemaphoreType.DMA,
        ],
        compiler_params=pltpu.CompilerParams(use_tc_tiling_on_sc=False),
    )
    def gk(idx_hbm, table_hbm, out_hbm, idx_v, rows_v, sem):
        wid = jax.lax.axis_index("s") * 2 + jax.lax.axis_index("c")
        base = wid * bpw
        pltpu.sync_copy(idx_hbm.at[pl.ds(base, bpw)], idx_v)
        pltpu.async_copy(table_hbm.at[idx_v], rows_v, sem).wait()
        pltpu.sync_copy(rows_v, out_hbm.at[pl.ds(base, bpw)])

    return gk(idx, table)


def kernel(x):
    xT = x.T
    grid = _N // _B

    ap_idx = jnp.concatenate([jnp.asarray(_A_IDX_NP), jnp.asarray(_P_IDX_NP)])
    ap_rows = _sc_gather(ap_idx, x, 2 * _N * (_K - 1))

    maxes = pl.pallas_call(
        _maxlw_kernel,
        grid=(grid,),
        in_specs=[
            pl.BlockSpec((_B, _D), lambda g: (g, 0)),
            pl.BlockSpec((_D, _N), lambda g: (0, 0)),
        ],
        out_specs=pl.BlockSpec((1, 1, 128), lambda g: (g, 0, 0)),
        out_shape=jax.ShapeDtypeStruct((grid, 1, 128), jnp.float32),
        interpret=_INTERPRET,
    )(x, xT)
    m = jnp.max(maxes).reshape(1, 1)

    idx_out = jax.ShapeDtypeStruct((1, _N), jnp.int32)
    o0, o1, o2 = pl.pallas_call(
        _sample_kernel,
        grid=(grid,),
        in_specs=[
            pl.BlockSpec(memory_space=pltpu.SMEM),
            pl.BlockSpec((_N, _D), lambda g: (0, 0)),
            pl.BlockSpec((_D, _B), lambda g: (0, g)),
        ],
        out_specs=[pl.BlockSpec((1, _B), lambda g: (0, g))] * 3,
        out_shape=[idx_out] * 3,
        scratch_shapes=[pltpu.VMEM((_N, _B), jnp.float32)],
        interpret=_INTERPRET,
    )(m, x, xT)
    n_idx = jnp.concatenate([o0, o1, o2], axis=0).T.reshape(-1)

    n_x = _sc_gather(n_idx, x, _N * (_K - 1))

    nt = _N * (_K - 1)
    a_x = ap_rows[:nt]
    p_x = ap_rows[nt:]
    return (jnp.asarray(_A_IDX_NP), a_x, p_x, n_x, x)
